# baseline (speedup 1.0000x reference)
import jax
import jax.numpy as jnp
from jax import lax
from jax.experimental import pallas as pl
from jax.experimental.pallas import tpu as pltpu
from jax.experimental.pallas import tpu_sc as plsc

D = 128
LANES = 16
DCHUNKS = D // LANES
NW = 32
PWIN = 64


def _embed_body(x_hbm, seg_hbm, tok_hbm, pos_hbm, segtab_hbm, out_hbm,
                ints_v, f32_v,
                tok_sems, idx_sem, seg_sem, segtab_sem, pos_sem, out_sem):
    nc = 2
    c = lax.axis_index("c")
    s = lax.axis_index("s")
    wid = s * nc + c
    batch = x_hbm.shape[0]
    pbase = wid * PWIN

    idx_v = ints_v.at[pl.ds(0, batch)]
    seg_v = ints_v.at[pl.ds(batch, batch)]
    rows_v = f32_v.at[pl.ds(0, batch * PWIN)]
    pos_v = f32_v.at[pl.ds(batch * PWIN, PWIN)]
    segtab_v = f32_v.at[pl.ds(batch * PWIN + PWIN, 2)]

    idx0_cp = pltpu.async_copy(x_hbm.at[0, pl.ds(pbase, PWIN)], idx_v.at[0],
                               tok_sems[0])
    idx_cps = [pltpu.async_copy(x_hbm.at[b, pl.ds(pbase, PWIN)], idx_v.at[b],
                                idx_sem) for b in range(1, batch)]
    pos_cp = pltpu.async_copy(pos_hbm.at[pl.ds(pbase, PWIN)], pos_v, pos_sem)
    seg_cps = [pltpu.async_copy(seg_hbm.at[b, pl.ds(pbase, PWIN)],
                                seg_v.at[b], seg_sem) for b in range(batch)]
    segtab_cp = pltpu.async_copy(segtab_hbm, segtab_v, segtab_sem)
    tok_cps = [None] * batch
    idx0_cp.wait()
    tok_cps[0] = pltpu.async_copy(tok_hbm.at[idx_v.at[0]],
                                  rows_v.at[pl.ds(0, PWIN)], tok_sems[0])
    for cp in idx_cps:
        cp.wait()
    for b in range(1, batch):
        tok_cps[b] = pltpu.async_copy(
            tok_hbm.at[idx_v.at[b]], rows_v.at[pl.ds(b * PWIN, PWIN)],
            tok_sems[b])
    for cp in seg_cps:
        cp.wait()
    segtab_cp.wait()

    seg0 = [segtab_v[0, pl.ds(j * LANES, LANES)] for j in range(DCHUNKS)]
    dif = [segtab_v[1, pl.ds(j * LANES, LANES)] - seg0[j] for j in range(DCHUNKS)]

    def make_grp_body(b):
        def grp_body(g, carry):
            sv = seg_v[b, pl.ds(g * LANES, LANES)].astype(jnp.float32)

            def tok_body(t, c2):
                i = b * PWIN + g * LANES + t
                ip = g * LANES + t
                sf = jnp.take(sv, jnp.full((LANES,), t, jnp.int32),
                              mode="fill")
                toks = [rows_v[i, pl.ds(j * LANES, LANES)]
                        for j in range(DCHUNKS)]
                poss = [pos_v[ip, pl.ds(j * LANES, LANES)]
                        for j in range(DCHUNKS)]
                for j in range(DCHUNKS):
                    rows_v[i, pl.ds(j * LANES, LANES)] = (
                        toks[j] + poss[j] + (seg0[j] + sf * dif[j]))
                return c2

            return lax.fori_loop(0, LANES, tok_body, carry)
        return grp_body

    pos_cp.wait()
    out_cps = []
    for b in range(batch):
        tok_cps[b].wait()
        lax.fori_loop(0, PWIN // LANES, make_grp_body(b), 0)
        out_cps.append(pltpu.async_copy(
            rows_v.at[pl.ds(b * PWIN, PWIN)],
            out_hbm.at[b, pl.ds(pbase, PWIN)], out_sem))
    for cp in out_cps:
        cp.wait()


def kernel(x, seg, tok_embed, pos_embed, seg_embed):
    batch, seq = x.shape
    mesh = plsc.VectorSubcoreMesh(core_axis_name="c", subcore_axis_name="s")
    out = pl.kernel(
        _embed_body,
        out_type=jax.ShapeDtypeStruct((batch, seq, D), jnp.float32),
        mesh=mesh,
        scratch_types=[
            pltpu.VMEM((2 * batch, PWIN), jnp.int32),
            pltpu.VMEM((batch * PWIN + PWIN + 2, D), jnp.float32),
            [pltpu.SemaphoreType.DMA] * batch,
            pltpu.SemaphoreType.DMA,
            pltpu.SemaphoreType.DMA,
            pltpu.SemaphoreType.DMA,
            pltpu.SemaphoreType.DMA,
            pltpu.SemaphoreType.DMA,
        ],
    )(x.astype(jnp.int32), seg.astype(jnp.int32), tok_embed, pos_embed,
      seg_embed)
    return out


# speedup vs baseline: 1.0177x; 1.0177x over previous
import jax
import jax.numpy as jnp
from jax import lax
from jax.experimental import pallas as pl
from jax.experimental.pallas import tpu as pltpu
from jax.experimental.pallas import tpu_sc as plsc

D = 128
LANES = 16
DCHUNKS = D // LANES
NW = 32
PWIN = 64


def _embed_body(x_hbm, seg_hbm, tok_hbm, pos_hbm, segtab_hbm, out_hbm,
                idx_v, seg_v, rows_v, pos_v, segtab_v,
                tok_sems, idx_sem, seg_sem, segtab_sem, pos_sem, out_sem):
    nc = 2
    c = lax.axis_index("c")
    s = lax.axis_index("s")
    wid = s * nc + c
    batch = x_hbm.shape[0]
    pbase = wid * PWIN

    idx0_cp = pltpu.async_copy(x_hbm.at[0, pl.ds(pbase, PWIN)], idx_v.at[0],
                               tok_sems[0])
    idx_cps = [pltpu.async_copy(x_hbm.at[b, pl.ds(pbase, PWIN)], idx_v.at[b],
                                idx_sem) for b in range(1, batch)]
    pos_cp = pltpu.async_copy(pos_hbm.at[pl.ds(pbase, PWIN)], pos_v, pos_sem)
    seg_cps = [pltpu.async_copy(seg_hbm.at[b, pl.ds(pbase, PWIN)],
                                seg_v.at[b], seg_sem) for b in range(batch)]
    segtab_cp = pltpu.async_copy(segtab_hbm, segtab_v, segtab_sem)
    tok_cps = [None] * batch
    idx0_cp.wait()
    tok_cps[0] = pltpu.async_copy(tok_hbm.at[idx_v.at[0]],
                                  rows_v.at[pl.ds(0, PWIN)], tok_sems[0])
    for cp in idx_cps:
        cp.wait()
    for b in range(1, batch):
        tok_cps[b] = pltpu.async_copy(
            tok_hbm.at[idx_v.at[b]], rows_v.at[pl.ds(b * PWIN, PWIN)],
            tok_sems[b])
    for cp in seg_cps:
        cp.wait()
    segtab_cp.wait()

    seg0 = [segtab_v[0, pl.ds(j * LANES, LANES)] for j in range(DCHUNKS)]
    dif = [segtab_v[1, pl.ds(j * LANES, LANES)] - seg0[j] for j in range(DCHUNKS)]

    def make_grp_body(b):
        def grp_body(g, carry):
            sv = seg_v[b, pl.ds(g * LANES, LANES)].astype(jnp.float32)

            def tok_body(t, c2):
                i = b * PWIN + g * LANES + t
                ip = g * LANES + t
                sf = jnp.take(sv, jnp.full((LANES,), t, jnp.int32),
                              mode="fill")
                toks = [rows_v[i, pl.ds(j * LANES, LANES)]
                        for j in range(DCHUNKS)]
                poss = [pos_v[ip, pl.ds(j * LANES, LANES)]
                        for j in range(DCHUNKS)]
                for j in range(DCHUNKS):
                    rows_v[i, pl.ds(j * LANES, LANES)] = (
                        toks[j] + poss[j] + (seg0[j] + sf * dif[j]))
                return c2

            return lax.fori_loop(0, LANES, tok_body, carry)
        return grp_body

    pos_cp.wait()
    out_cps = []
    for b in range(batch):
        tok_cps[b].wait()
        lax.fori_loop(0, PWIN // LANES, make_grp_body(b), 0)
        out_cps.append(pltpu.async_copy(
            rows_v.at[pl.ds(b * PWIN, PWIN)],
            out_hbm.at[b, pl.ds(pbase, PWIN)], out_sem))
    for cp in out_cps:
        cp.wait()


def kernel(x, seg, tok_embed, pos_embed, seg_embed):
    batch, seq = x.shape
    mesh = plsc.VectorSubcoreMesh(core_axis_name="c", subcore_axis_name="s")
    out = pl.kernel(
        _embed_body,
        out_type=jax.ShapeDtypeStruct((batch, seq, D), jnp.float32),
        mesh=mesh,
        scratch_types=[
            pltpu.VMEM((batch, PWIN), jnp.int32),
            pltpu.VMEM((batch, PWIN), jnp.int32),
            pltpu.VMEM((batch * PWIN, D), jnp.float32),
            pltpu.VMEM((PWIN, D), jnp.float32),
            pltpu.VMEM((2, D), jnp.float32),
            [pltpu.SemaphoreType.DMA] * batch,
            pltpu.SemaphoreType.DMA,
            pltpu.SemaphoreType.DMA,
            pltpu.SemaphoreType.DMA,
            pltpu.SemaphoreType.DMA,
            pltpu.SemaphoreType.DMA,
        ],
    )(x.astype(jnp.int32), seg.astype(jnp.int32), tok_embed, pos_embed,
      seg_embed)
    return out


# speedup vs baseline: 1.0235x; 1.0057x over previous
import jax
import jax.numpy as jnp
from jax import lax
from jax.experimental import pallas as pl
from jax.experimental.pallas import tpu as pltpu
from jax.experimental.pallas import tpu_sc as plsc

D = 128
LANES = 16
DCHUNKS = D // LANES
NW = 32
PWIN = 64


def _embed_body(x_hbm, seg_hbm, tok_hbm, pos_hbm, segtab_hbm, out_hbm,
                idx_v, seg_v, rows_v, pos_v, segtab_v,
                tok_sems, grp_sems, idx_sem, seg_sem, segtab_sem, pos_sem,
                out_sem):
    nc = 2
    c = lax.axis_index("c")
    s = lax.axis_index("s")
    wid = s * nc + c
    batch = x_hbm.shape[0]
    pbase = wid * PWIN

    idx0_cp = pltpu.async_copy(x_hbm.at[0, pl.ds(pbase, PWIN)], idx_v.at[0],
                               tok_sems[0])
    idx_cps = [pltpu.async_copy(x_hbm.at[b, pl.ds(pbase, PWIN)], idx_v.at[b],
                                idx_sem) for b in range(1, batch)]
    pos_cp = pltpu.async_copy(pos_hbm.at[pl.ds(pbase, PWIN)], pos_v, pos_sem)
    seg_cps = [pltpu.async_copy(seg_hbm.at[b, pl.ds(pbase, PWIN)],
                                seg_v.at[b], seg_sem) for b in range(batch)]
    segtab_cp = pltpu.async_copy(segtab_hbm, segtab_v, segtab_sem)
    tok_cps = [None] * batch
    idx0_cp.wait()
    grp_cps = [pltpu.async_copy(
        tok_hbm.at[idx_v.at[0].at[pl.ds(g * LANES, LANES)]],
        rows_v.at[pl.ds(g * LANES, LANES)], grp_sems[g])
        for g in range(PWIN // LANES)]
    for cp in idx_cps:
        cp.wait()
    for b in range(1, batch):
        tok_cps[b] = pltpu.async_copy(
            tok_hbm.at[idx_v.at[b]], rows_v.at[pl.ds(b * PWIN, PWIN)],
            tok_sems[b])
    for cp in seg_cps:
        cp.wait()
    segtab_cp.wait()

    seg0 = [segtab_v[0, pl.ds(j * LANES, LANES)] for j in range(DCHUNKS)]
    dif = [segtab_v[1, pl.ds(j * LANES, LANES)] - seg0[j] for j in range(DCHUNKS)]

    def make_grp_body(b):
        def grp_body(g, carry):
            sv = seg_v[b, pl.ds(g * LANES, LANES)].astype(jnp.float32)

            def tok_body(t, c2):
                i = b * PWIN + g * LANES + t
                ip = g * LANES + t
                sf = jnp.take(sv, jnp.full((LANES,), t, jnp.int32),
                              mode="fill")
                toks = [rows_v[i, pl.ds(j * LANES, LANES)]
                        for j in range(DCHUNKS)]
                poss = [pos_v[ip, pl.ds(j * LANES, LANES)]
                        for j in range(DCHUNKS)]
                for j in range(DCHUNKS):
                    rows_v[i, pl.ds(j * LANES, LANES)] = (
                        toks[j] + poss[j] + (seg0[j] + sf * dif[j]))
                return c2

            return lax.fori_loop(0, LANES, tok_body, carry)
        return grp_body

    pos_cp.wait()
    out_cps = []
    body0 = make_grp_body(0)
    for g in range(PWIN // LANES):
        grp_cps[g].wait()
        body0(g, 0)
    out_cps.append(pltpu.async_copy(
        rows_v.at[pl.ds(0, PWIN)],
        out_hbm.at[0, pl.ds(pbase, PWIN)], out_sem))
    for b in range(1, batch):
        tok_cps[b].wait()
        lax.fori_loop(0, PWIN // LANES, make_grp_body(b), 0)
        out_cps.append(pltpu.async_copy(
            rows_v.at[pl.ds(b * PWIN, PWIN)],
            out_hbm.at[b, pl.ds(pbase, PWIN)], out_sem))
    for cp in out_cps:
        cp.wait()


def kernel(x, seg, tok_embed, pos_embed, seg_embed):
    batch, seq = x.shape
    mesh = plsc.VectorSubcoreMesh(core_axis_name="c", subcore_axis_name="s")
    out = pl.kernel(
        _embed_body,
        out_type=jax.ShapeDtypeStruct((batch, seq, D), jnp.float32),
        mesh=mesh,
        scratch_types=[
            pltpu.VMEM((batch, PWIN), jnp.int32),
            pltpu.VMEM((batch, PWIN), jnp.int32),
            pltpu.VMEM((batch * PWIN, D), jnp.float32),
            pltpu.VMEM((PWIN, D), jnp.float32),
            pltpu.VMEM((2, D), jnp.float32),
            [pltpu.SemaphoreType.DMA] * batch,
            [pltpu.SemaphoreType.DMA] * (PWIN // LANES),
            pltpu.SemaphoreType.DMA,
            pltpu.SemaphoreType.DMA,
            pltpu.SemaphoreType.DMA,
            pltpu.SemaphoreType.DMA,
            pltpu.SemaphoreType.DMA,
        ],
    )(x.astype(jnp.int32), seg.astype(jnp.int32), tok_embed, pos_embed,
      seg_embed)
    return out
